# baseline (speedup 1.0000x reference)
import jax
import jax.numpy as jnp
from jax import lax
from jax.experimental import pallas as pl
from jax.experimental.pallas import tpu as pltpu

_LAT, _K = 64, 1024
_BLK = 448


def _conv2d(x, w, b, stride, pad):
    out = lax.conv_general_dilated(
        x, w, (stride, stride), [(pad, pad), (pad, pad)],
        dimension_numbers=('NCHW', 'OIHW', 'NCHW'))
    return out + b[None, :, None, None]


def _conv_t2d(x, w, b, stride, k):
    w2 = jnp.flip(w, axis=(2, 3)).transpose(1, 0, 2, 3)
    out = lax.conv_general_dilated(
        x, w2, (1, 1), [(k - 1, k - 1), (k - 1, k - 1)],
        lhs_dilation=(stride, stride),
        dimension_numbers=('NCHW', 'OIHW', 'NCHW'))
    return out + b[None, :, None, None]


def _conv2d_fast(x, w, b, stride, pad):
    wt = w.transpose(2, 3, 1, 0)
    out = lax.conv_general_dilated(
        x.astype(jnp.bfloat16), wt.astype(jnp.bfloat16), (stride, stride),
        [(pad, pad), (pad, pad)],
        dimension_numbers=('NHWC', 'HWIO', 'NHWC'),
        preferred_element_type=jnp.float32)
    return out + b[None, None, None, :]


def _conv_t2d_fast(x, w, b, stride, k):
    wt = jnp.flip(w, axis=(2, 3)).transpose(2, 3, 0, 1)
    out = lax.conv_general_dilated(
        x.astype(jnp.bfloat16), wt.astype(jnp.bfloat16), (1, 1),
        [(k - 1, k - 1), (k - 1, k - 1)],
        lhs_dilation=(stride, stride),
        dimension_numbers=('NHWC', 'HWIO', 'NHWC'),
        preferred_element_type=jnp.float32)
    return out + b[None, None, None, :]


def _batchnorm_nhwc(x, g, b, eps=1e-5):
    m = jnp.mean(x, axis=(0, 1, 2), keepdims=True)
    v = jnp.var(x, axis=(0, 1, 2), keepdims=True)
    return g[None, None, None, :] * (x - m) / jnp.sqrt(v + eps) + b[None, None, None, :]


def _batchnorm(x, g, b, eps=1e-5):
    m = jnp.mean(x, axis=(0, 2, 3), keepdims=True)
    v = jnp.var(x, axis=(0, 2, 3), keepdims=True)
    return g[None, :, None, None] * (x - m) / jnp.sqrt(v + eps) + b[None, :, None, None]


def _leaky(x):
    return jnp.where(x >= 0, x, 0.01 * x)


def _vq_body(z_ref, cb_ref, idx_ref, quant_ref, rloss_ref):
    z = z_ref[...]
    cb = cb_ref[...]
    s = lax.dot_general(z, cb, (((1,), (1,)), ((), ())),
                        preferred_element_type=jnp.float32)
    zsq = jnp.sum(z * z, axis=1, keepdims=True)
    cbsq = jnp.sum(cb * cb, axis=1)
    d2 = zsq - 2.0 * s + cbsq[None, :]
    dist = jnp.sqrt(jnp.maximum(d2, 0.0))
    m = jnp.min(dist, axis=1, keepdims=True)
    ids = lax.broadcasted_iota(jnp.int32, (_BLK, _K), 1)
    idx = jnp.min(jnp.where(dist == m, ids, _K), axis=1)
    idx_ref[0, 0, :] = idx
    onehot = (ids == idx[:, None]).astype(jnp.float32)
    quant = lax.dot_general(onehot, cb, (((1,), (0,)), ((), ())),
                            preferred_element_type=jnp.float32)
    quant_ref[...] = quant
    r = z - quant
    rloss_ref[0, 0, :] = jnp.sum(r * r, axis=1)


def _vq(z, cb):
    rows = z.shape[0]
    nblk = rows // _BLK
    idx3, quant, rloss = pl.pallas_call(
        _vq_body,
        grid=(nblk,),
        in_specs=[
            pl.BlockSpec((_BLK, _LAT), lambda i: (i, 0)),
            pl.BlockSpec((_K, _LAT), lambda i: (0, 0)),
        ],
        out_specs=[
            pl.BlockSpec((1, 1, _BLK), lambda i: (i, 0, 0)),
            pl.BlockSpec((_BLK, _LAT), lambda i: (i, 0)),
            pl.BlockSpec((1, 1, _BLK), lambda i: (i, 0, 0)),
        ],
        out_shape=[
            jax.ShapeDtypeStruct((nblk, 1, _BLK), jnp.int32),
            jax.ShapeDtypeStruct((rows, _LAT), jnp.float32),
            jax.ShapeDtypeStruct((nblk, 1, _BLK), jnp.float32),
        ],
    )(z, cb)
    idx = idx3.reshape(rows)
    loss = jnp.sum(rloss) / (rows * _LAT)
    return idx, quant, loss


def kernel(x, params):
    p = params
    out = _leaky(_batchnorm(_conv2d(x, p['enc_w0'], p['enc_b0'], 2, 1),
                            p['enc_g0'], p['enc_be0']))
    out = _leaky(_batchnorm(_conv2d(out, p['enc_w1'], p['enc_b1'], 2, 1),
                            p['enc_g1'], p['enc_be1']))
    out = _conv2d(out, p['enc_w2'], p['enc_b2'], 2, 1)
    out = _conv2d(out, p['preq_w'], p['preq_b'], 1, 0)
    Bn, lat, H, W = out.shape
    z = out.transpose(0, 2, 3, 1).reshape(Bn * H * W, lat)
    idx, quant, loss = _vq(z, p['codebook'])
    idx = idx.reshape(Bn, H, W)
    quant = quant.reshape(Bn, H, W, lat)
    out = _conv2d_fast(quant, p['postq_w'], p['postq_b'], 1, 0)
    out = _leaky(_batchnorm_nhwc(_conv_t2d_fast(out, p['dec_w0'], p['dec_b0'], 2, 4),
                                 p['dec_g0'], p['dec_be0']))
    out = _leaky(_batchnorm_nhwc(_conv_t2d_fast(out, p['dec_w1'], p['dec_b1'], 2, 4),
                                 p['dec_g1'], p['dec_be1']))
    out = jnp.tanh(_conv_t2d_fast(out, p['dec_w2'], p['dec_b2'], 2, 4))
    out = out.transpose(0, 3, 1, 2)
    return (out, idx, loss, loss)


# speedup vs baseline: 1.2335x; 1.2335x over previous
import functools

import jax
import jax.numpy as jnp
from jax import lax
from jax.experimental import pallas as pl
from jax.experimental.pallas import tpu as pltpu
from jax.experimental.pallas import tpu_sc as plsc

_LAT, _K = 64, 1024
_BLK = 448
_NW = 32
_GPAD = 6400


def _conv2d(x, w, b, stride, pad):
    out = lax.conv_general_dilated(
        x, w, (stride, stride), [(pad, pad), (pad, pad)],
        dimension_numbers=('NCHW', 'OIHW', 'NCHW'))
    return out + b[None, :, None, None]


def _conv_t2d(x, w, b, stride, k):
    w2 = jnp.flip(w, axis=(2, 3)).transpose(1, 0, 2, 3)
    out = lax.conv_general_dilated(
        x, w2, (1, 1), [(k - 1, k - 1), (k - 1, k - 1)],
        lhs_dilation=(stride, stride),
        dimension_numbers=('NCHW', 'OIHW', 'NCHW'))
    return out + b[None, :, None, None]


def _conv2d_fast(x, w, b, stride, pad):
    out = lax.conv_general_dilated(
        x.astype(jnp.bfloat16), w.astype(jnp.bfloat16), (stride, stride),
        [(pad, pad), (pad, pad)],
        dimension_numbers=('NCHW', 'OIHW', 'NCHW'),
        preferred_element_type=jnp.float32)
    return out + b[None, :, None, None]


def _conv_t2d_fast(x, w, b, stride, k):
    w2 = jnp.flip(w, axis=(2, 3)).transpose(1, 0, 2, 3)
    out = lax.conv_general_dilated(
        x.astype(jnp.bfloat16), w2.astype(jnp.bfloat16), (1, 1),
        [(k - 1, k - 1), (k - 1, k - 1)],
        lhs_dilation=(stride, stride),
        dimension_numbers=('NCHW', 'OIHW', 'NCHW'),
        preferred_element_type=jnp.float32)
    return out + b[None, :, None, None]


def _batchnorm(x, g, b, eps=1e-5):
    m = jnp.mean(x, axis=(0, 2, 3), keepdims=True)
    v = jnp.var(x, axis=(0, 2, 3), keepdims=True)
    return g[None, :, None, None] * (x - m) / jnp.sqrt(v + eps) + b[None, :, None, None]


def _leaky(x):
    return jnp.where(x >= 0, x, 0.01 * x)


_DH = 118
_DHP = _DH + 1
_DFLAT = 120 * 128


def _dec2_body(ypf_ref, w4_ref, b_ref, e_ref, out_ref):
    yp = ypf_ref[0]
    y0 = yp[:, 128:_DFLAT]
    y1 = yp[:, 0:_DFLAT - 128]
    dn = (((1,), (0,)), ((), ()))
    acc0 = (lax.dot_general(w4_ref[0, 0], y0, dn, preferred_element_type=jnp.float32)
            + lax.dot_general(w4_ref[1, 0], y1, dn, preferred_element_type=jnp.float32))
    acc1 = (lax.dot_general(w4_ref[0, 1], y0, dn, preferred_element_type=jnp.float32)
            + lax.dot_general(w4_ref[1, 1], y1, dn, preferred_element_type=jnp.float32))
    shifted = jnp.concatenate(
        [acc0[:, 1:], jnp.zeros((16, 1), jnp.float32)], axis=1)
    zt = jnp.tanh(acc1 + shifted + b_ref[:, :1])
    z4 = zt.reshape(16, _DHP, 128)
    for a in (0, 1):
        res = jnp.zeros((4 * _DHP, 256), jnp.float32)
        for c in (0, 1):
            zac = z4[(a * 2 + c) * 4:(a * 2 + c) * 4 + 4].reshape(4 * _DHP, 128)
            res = res + lax.dot_general(zac, e_ref[c], dn,
                                        preferred_element_type=jnp.float32)
        out_ref[0, a] = res.reshape(4, _DHP, 256)


def _dec2_tanh(y, w, b):
    n = y.shape[0]
    ypf = jnp.pad(y, ((0, 0), (0, 0), (1, 1), (1, 9))).reshape(n, 64, _DFLAT)
    t = w.reshape(64, 3, 2, 2, 2, 2)
    t = t.transpose(2, 4, 3, 5, 1, 0)
    t = jnp.pad(t, ((0, 0),) * 4 + ((0, 1), (0, 0)))
    w4 = t.reshape(2, 2, 16, 64)
    b16 = jnp.broadcast_to(jnp.tile(jnp.pad(b, (0, 1)), 4)[:, None], (16, 128))
    s_i = jnp.arange(128)[:, None]
    q_i = jnp.arange(256)[None, :]
    e = jnp.stack([((q_i == 2 * s_i + c) & (s_i < _DHP)).astype(jnp.float32)
                   for c in (0, 1)])
    res = pl.pallas_call(
        _dec2_body,
        grid=(n,),
        in_specs=[
            pl.BlockSpec((1, 64, _DFLAT), lambda i: (i, 0, 0)),
            pl.BlockSpec((2, 2, 16, 64), lambda i: (0, 0, 0, 0)),
            pl.BlockSpec((16, 128), lambda i: (0, 0)),
            pl.BlockSpec((2, 128, 256), lambda i: (0, 0, 0)),
        ],
        out_specs=pl.BlockSpec((1, 2, 4, _DHP, 256), lambda i: (i, 0, 0, 0, 0)),
        out_shape=jax.ShapeDtypeStruct((n, 2, 4, _DHP, 256), jnp.float32),
    )(ypf, w4, b16, e)
    out = res[:, :, :3, :, :238]
    return out.transpose(0, 2, 3, 1, 4).reshape(n, 3, 238, 238)


def _argmin_body(z_ref, cb_ref, idx_ref):
    z = z_ref[...]
    cb = cb_ref[...]
    s = lax.dot_general(z, cb, (((1,), (1,)), ((), ())),
                        preferred_element_type=jnp.float32)
    zsq = jnp.sum(z * z, axis=1, keepdims=True)
    cbsq = jnp.sum(cb * cb, axis=1)
    d2 = zsq - 2.0 * s + cbsq[None, :]
    dist = jnp.sqrt(jnp.maximum(d2, 0.0))
    m = jnp.min(dist, axis=1, keepdims=True)
    ids = lax.broadcasted_iota(jnp.int32, (_BLK, _K), 1)
    idx_ref[0, 0, :] = jnp.min(jnp.where(dist == m, ids, _K), axis=1)


def _loss_body(z_ref, q_ref, rloss_ref):
    r = z_ref[...] - q_ref[...]
    rloss_ref[0, 0, :] = jnp.sum(r * r, axis=1)


@functools.lru_cache(maxsize=1)
def _make_sc_gather():
    @functools.partial(
        pl.kernel,
        out_type=jax.ShapeDtypeStruct((_GPAD, 128), jnp.float32),
        mesh=plsc.VectorSubcoreMesh(core_axis_name="c", subcore_axis_name="s"),
        scratch_types=[
            pltpu.VMEM((_GPAD // _NW,), jnp.int32),
            pltpu.VMEM((_GPAD // _NW, 128), jnp.float32),
            pltpu.SemaphoreType.DMA,
        ],
    )
    def _sc_gather(cb_hbm, idx_hbm, out_hbm, idx_v, rows_v, sem):
        bpw = _GPAD // _NW
        wid = lax.axis_index("s") * 2 + lax.axis_index("c")
        base = wid * bpw
        pltpu.sync_copy(idx_hbm.at[pl.ds(base, bpw)], idx_v)
        pltpu.async_copy(cb_hbm.at[idx_v], rows_v, sem).wait()
        pltpu.sync_copy(rows_v, out_hbm.at[pl.ds(base, bpw)])

    return _sc_gather


def _vq(z, cb):
    rows = z.shape[0]
    nblk = rows // _BLK
    idx3 = pl.pallas_call(
        _argmin_body,
        grid=(nblk,),
        in_specs=[
            pl.BlockSpec((_BLK, _LAT), lambda i: (i, 0)),
            pl.BlockSpec((_K, _LAT), lambda i: (0, 0)),
        ],
        out_specs=pl.BlockSpec((1, 1, _BLK), lambda i: (i, 0, 0)),
        out_shape=jax.ShapeDtypeStruct((nblk, 1, _BLK), jnp.int32),
    )(z, cb)
    idx = idx3.reshape(rows)
    idx_pad = jnp.concatenate([idx, jnp.zeros((_GPAD - rows,), jnp.int32)])
    cb_pad = jnp.pad(cb, ((0, 0), (0, 128 - _LAT)))
    quant = _make_sc_gather()(cb_pad, idx_pad)[:rows, :_LAT]
    rloss = pl.pallas_call(
        _loss_body,
        grid=(nblk,),
        in_specs=[
            pl.BlockSpec((_BLK, _LAT), lambda i: (i, 0)),
            pl.BlockSpec((_BLK, _LAT), lambda i: (i, 0)),
        ],
        out_specs=pl.BlockSpec((1, 1, _BLK), lambda i: (i, 0, 0)),
        out_shape=jax.ShapeDtypeStruct((nblk, 1, _BLK), jnp.float32),
    )(z, quant)
    loss = jnp.sum(rloss) / (rows * _LAT)
    return idx, quant, loss


def kernel(x, params):
    p = params
    out = _leaky(_batchnorm(_conv2d(x, p['enc_w0'], p['enc_b0'], 2, 1),
                            p['enc_g0'], p['enc_be0']))
    out = _leaky(_batchnorm(_conv2d(out, p['enc_w1'], p['enc_b1'], 2, 1),
                            p['enc_g1'], p['enc_be1']))
    out = _conv2d(out, p['enc_w2'], p['enc_b2'], 2, 1)
    out = _conv2d(out, p['preq_w'], p['preq_b'], 1, 0)
    Bn, lat, H, W = out.shape
    z = out.transpose(0, 2, 3, 1).reshape(Bn * H * W, lat)
    idx, quant, loss = _vq(z, p['codebook'])
    idx = idx.reshape(Bn, H, W)
    quant = quant.reshape(Bn, H, W, lat).transpose(0, 3, 1, 2)
    out = _conv2d_fast(quant, p['postq_w'], p['postq_b'], 1, 0)
    out = _leaky(_batchnorm(_conv_t2d_fast(out, p['dec_w0'], p['dec_b0'], 2, 4),
                            p['dec_g0'], p['dec_be0']))
    out = _leaky(_batchnorm(_conv_t2d_fast(out, p['dec_w1'], p['dec_b1'], 2, 4),
                            p['dec_g1'], p['dec_be1']))
    out = _dec2_tanh(out, p['dec_w2'], p['dec_b2'])
    return (out, idx, loss, loss)
